# baseline (speedup 1.0000x reference)
import functools

import jax
import jax.numpy as jnp
from jax.experimental import pallas as pl


def _ncm_body(x_ref, mu_ref, ck_ref, out_ref):
    x = x_ref[...]
    mu = mu_ref[...]
    ck = ck_ref[...]

    xn = jnp.sum(x * x, axis=1, keepdims=True)
    ones_row = jnp.ones((1, x.shape[1]), jnp.float32)
    mn = jax.lax.dot_general(
        ones_row, mu * mu,
        dimension_numbers=(((1,), (1,)), ((), ())),
        preferred_element_type=jnp.float32,
    )

    g = jax.lax.dot_general(
        x, mu,
        dimension_numbers=(((1,), (1,)), ((), ())),
        preferred_element_type=jnp.float32,
    )

    scores = 2.0 * g - xn - mn
    min_col = jnp.min(scores, axis=1, keepdims=True) - 1.0
    out_ref[...] = jnp.where(ck == 0.0, min_col, scores)


@jax.jit
def kernel(X, muK, cK):
    m, d = X.shape
    k = muK.shape[0]
    ck2 = cK.reshape(1, k)
    bm = 128
    return pl.pallas_call(
        _ncm_body,
        grid=(m // bm,),
        in_specs=[
            pl.BlockSpec((bm, d), lambda i: (i, 0)),
            pl.BlockSpec((k, d), lambda i: (0, 0)),
            pl.BlockSpec((1, k), lambda i: (0, 0)),
        ],
        out_specs=pl.BlockSpec((bm, k), lambda i: (i, 0)),
        out_shape=jax.ShapeDtypeStruct((m, k), jnp.float32),
    )(X, muK, ck2)


# speedup vs baseline: 1.2962x; 1.2962x over previous
import functools

import jax
import jax.numpy as jnp
from jax.experimental import pallas as pl


def _ncm_body(x_ref, mu_ref, ck_ref, out_ref):
    x = x_ref[...]
    mu = mu_ref[...]
    ck = ck_ref[...]

    xn = jnp.sum(x * x, axis=1, keepdims=True)
    ones_row = jnp.ones((1, x.shape[1]), jnp.float32)
    mn = jax.lax.dot_general(
        ones_row, mu * mu,
        dimension_numbers=(((1,), (1,)), ((), ())),
        preferred_element_type=jnp.float32,
    )

    g = jax.lax.dot_general(
        x, mu,
        dimension_numbers=(((1,), (1,)), ((), ())),
        preferred_element_type=jnp.float32,
    )

    scores = 2.0 * g - xn - mn
    min_col = jnp.min(scores, axis=1, keepdims=True) - 1.0
    out_ref[...] = jnp.where(ck == 0.0, min_col, scores)


@jax.jit
def kernel(X, muK, cK):
    m, d = X.shape
    k = muK.shape[0]
    ck2 = cK.reshape(1, k)
    bm = m
    return pl.pallas_call(
        _ncm_body,
        grid=(m // bm,),
        in_specs=[
            pl.BlockSpec((bm, d), lambda i: (i, 0)),
            pl.BlockSpec((k, d), lambda i: (0, 0)),
            pl.BlockSpec((1, k), lambda i: (0, 0)),
        ],
        out_specs=pl.BlockSpec((bm, k), lambda i: (i, 0)),
        out_shape=jax.ShapeDtypeStruct((m, k), jnp.float32),
    )(X, muK, ck2)


# speedup vs baseline: 1.3464x; 1.0388x over previous
import functools

import jax
import jax.numpy as jnp
from jax.experimental import pallas as pl


def _ncm_body(x_ref, mu_ref, ck_ref, out_ref):
    x = x_ref[...]
    mu = mu_ref[...]
    ck = ck_ref[...]

    xn = jnp.sum(x * x, axis=1, keepdims=True)
    ones_row = jnp.ones((1, x.shape[1]), jnp.float32)
    mn = jax.lax.dot_general(
        ones_row, mu * mu,
        dimension_numbers=(((1,), (1,)), ((), ())),
        preferred_element_type=jnp.float32,
    )

    g = jax.lax.dot_general(
        x, mu,
        dimension_numbers=(((1,), (1,)), ((), ())),
        preferred_element_type=jnp.float32,
    )

    scores = 2.0 * g - xn - mn
    min_col = jnp.min(scores, axis=1, keepdims=True) - 1.0
    out_ref[...] = jnp.where(ck == 0.0, min_col, scores)


@jax.jit
def kernel(X, muK, cK):
    m, d = X.shape
    k = muK.shape[0]
    ck2 = cK.reshape(1, k)
    bm = 512
    return pl.pallas_call(
        _ncm_body,
        grid=(m // bm,),
        in_specs=[
            pl.BlockSpec((bm, d), lambda i: (i, 0)),
            pl.BlockSpec((k, d), lambda i: (0, 0)),
            pl.BlockSpec((1, k), lambda i: (0, 0)),
        ],
        out_specs=pl.BlockSpec((bm, k), lambda i: (i, 0)),
        out_shape=jax.ShapeDtypeStruct((m, k), jnp.float32),
    )(X, muK, ck2)
